# Initial kernel scaffold; baseline (speedup 1.0000x reference)
import jax
import jax.numpy as jnp
from jax.experimental import pallas as pl


def kernel(x, edge_index, W1, b1, W2, b2, eps):
    raise NotImplementedError("write your pallas kernel here")


# speedup vs baseline: 7.7604x; 7.7604x over previous
import functools

import jax
import jax.numpy as jnp
from jax import lax
from jax.experimental import pallas as pl
from jax.experimental.pallas import tpu as pltpu
from jax.experimental.pallas import tpu_sc as plsc

NC = 2
NS = 16
NW = NC * NS
CHUNK = 80


def _sc_aggregate(x, src, dst, n_chunk, npad, nps, zrows):
    _, d = x.shape
    mesh = plsc.VectorSubcoreMesh(core_axis_name="c", subcore_axis_name="s")

    @functools.partial(
        pl.kernel,
        mesh=mesh,
        out_type=jax.ShapeDtypeStruct((NC, npad, d), jnp.float32),
        scratch_types=[
            pltpu.VMEM((n_chunk, CHUNK), jnp.int32),
            pltpu.VMEM((n_chunk, CHUNK), jnp.int32),
            pltpu.VMEM((CHUNK, d), jnp.float32),
            pltpu.VMEM((zrows, d), jnp.float32),
            pltpu.VMEM_SHARED((npad, d), jnp.float32),
        ],
    )
    def agg_kernel(x_hbm, src_hbm, dst_hbm, out_hbm, src_v, dst_v, rows_v,
                   zbuf, agg_sh):
        cid = lax.axis_index("c")
        sid = lax.axis_index("s")
        wid = sid * NC + cid

        def zero_row(r, carry):
            for j in range(d // 16):
                zbuf[r, pl.ds(j * 16, 16)] = jnp.zeros((16,), jnp.float32)
            return carry
        lax.fori_loop(0, zrows, zero_row, 0)
        row0 = sid * nps
        for k in range(nps // zrows):
            pltpu.sync_copy(zbuf, agg_sh.at[pl.ds(row0 + k * zrows, zrows)])
        plsc.subcore_barrier()

        pltpu.sync_copy(src_hbm.at[wid], src_v)
        pltpu.sync_copy(dst_hbm.at[wid], dst_v)

        def body(c, carry):
            pltpu.sync_copy(x_hbm.at[src_v.at[c]], rows_v)
            pltpu.sync_copy(rows_v, agg_sh.at[dst_v.at[c]], add=True)
            return carry
        lax.fori_loop(0, n_chunk, body, 0)
        plsc.subcore_barrier()

        pltpu.sync_copy(agg_sh.at[pl.ds(row0, nps)],
                        out_hbm.at[cid, pl.ds(row0, nps)])

    return agg_kernel(x, src, dst)


def _mlp_body(eps_ref, x_ref, agg_ref, w1_ref, b1_ref, w2_ref, b2_ref, o_ref):
    h = x_ref[...] * (1.0 + eps_ref[0]) + agg_ref[0] + agg_ref[1]
    h = jnp.dot(h, w1_ref[...], preferred_element_type=jnp.float32) + b1_ref[...]
    h = jnp.maximum(h, 0.0)
    o_ref[...] = (jnp.dot(h, w2_ref[...], preferred_element_type=jnp.float32)
                  + b2_ref[...])


def kernel(x, edge_index, W1, b1, W2, b2, eps):
    n, d = x.shape
    e = edge_index.shape[1]
    e_per_w = e // NW
    n_chunk = e_per_w // CHUNK
    npad = 10240
    nps = npad // NS
    zrows = 16

    src = edge_index[0].reshape(NW, n_chunk, CHUNK)
    dst = edge_index[1].reshape(NW, n_chunk, CHUNK)
    partial = _sc_aggregate(x, src, dst, n_chunk, npad, nps, zrows)

    br = 2000
    grid = (n // br,)
    out = pl.pallas_call(
        _mlp_body,
        grid=grid,
        in_specs=[
            pl.BlockSpec(memory_space=pltpu.SMEM),
            pl.BlockSpec((br, d), lambda i: (i, 0)),
            pl.BlockSpec((NC, br, d), lambda i: (0, i, 0)),
            pl.BlockSpec((d, d), lambda i: (0, 0)),
            pl.BlockSpec((1, d), lambda i: (0, 0)),
            pl.BlockSpec((d, d), lambda i: (0, 0)),
            pl.BlockSpec((1, d), lambda i: (0, 0)),
        ],
        out_specs=pl.BlockSpec((br, d), lambda i: (i, 0)),
        out_shape=jax.ShapeDtypeStruct((n, d), jnp.float32),
    )(eps.reshape(1), x, partial, W1, b1.reshape(1, d), W2, b2.reshape(1, d))
    return out


# speedup vs baseline: 10.1823x; 1.3121x over previous
import functools

import jax
import jax.numpy as jnp
from jax import lax
from jax.experimental import pallas as pl
from jax.experimental.pallas import tpu as pltpu
from jax.experimental.pallas import tpu_sc as plsc

NC = 2
NS = 16
NW = NC * NS
CHUNK = 40
NBUF = 4
DEPTH = 2
G = 25
ZR = 16


def _sc_aggregate(x, src, dst, n_chunk, npad, nps):
    _, d = x.shape
    mesh = plsc.VectorSubcoreMesh(core_axis_name="c", subcore_axis_name="s")

    @functools.partial(
        pl.kernel,
        mesh=mesh,
        out_type=jax.ShapeDtypeStruct((NC, npad, d), jnp.float32),
        scratch_types=[
            pltpu.VMEM((2, G, CHUNK), jnp.int32),
            pltpu.VMEM((2, G, CHUNK), jnp.int32),
            pltpu.VMEM((NBUF, CHUNK, d), jnp.float32),
            pltpu.VMEM((ZR, d), jnp.float32),
            pltpu.VMEM_SHARED((npad, d), jnp.float32),
            pltpu.SemaphoreType.DMA((NBUF,)),
            pltpu.SemaphoreType.DMA((NBUF,)),
            pltpu.SemaphoreType.DMA,
        ],
    )
    def agg_kernel(x_hbm, src_hbm, dst_hbm, out_hbm, srcb, dstb, rows, zbuf,
                   agg_sh, gsem, ssem, zsem):
        cid = lax.axis_index("c")
        sid = lax.axis_index("s")
        wid = sid * NC + cid
        row0 = sid * nps

        pltpu.sync_copy(src_hbm.at[wid, 0], srcb.at[0])
        pltpu.sync_copy(dst_hbm.at[wid, 0], dstb.at[0])
        for k in range(DEPTH):
            pltpu.async_copy(x_hbm.at[srcb.at[0, k]], rows.at[k], gsem.at[k])

        def zero_row(r, carry):
            for j in range(d // 16):
                zbuf[r, pl.ds(j * 16, 16)] = jnp.zeros((16,), jnp.float32)
            return carry
        lax.fori_loop(0, ZR, zero_row, 0)
        for k in range(nps // ZR):
            pltpu.async_copy(zbuf, agg_sh.at[pl.ds(row0 + k * ZR, ZR)], zsem)
        for k in range(nps // ZR):
            pltpu.make_async_copy(zbuf, agg_sh.at[pl.ds(row0, ZR)], zsem).wait()
        plsc.subcore_barrier()

        def body(c, carry):
            nxt = c + DEPTH + 1
            @pl.when(jnp.logical_and(lax.rem(nxt, G) == 0, nxt < n_chunk))
            def _():
                g = nxt // G
                slot = lax.rem(g, 2)
                pltpu.sync_copy(src_hbm.at[wid, g], srcb.at[slot])
                pltpu.sync_copy(dst_hbm.at[wid, g], dstb.at[slot])

            buf = lax.rem(c, NBUF)
            slot_c = lax.rem(c // G, 2)
            pos_c = lax.rem(c, G)
            pltpu.make_async_copy(x_hbm.at[srcb.at[slot_c, pos_c]],
                                  rows.at[buf], gsem.at[buf]).wait()
            pltpu.async_copy(rows.at[buf], agg_sh.at[dstb.at[slot_c, pos_c]],
                             ssem.at[buf], add=True)
            j = c + DEPTH
            @pl.when(j < n_chunk)
            def _():
                jbuf = lax.rem(j, NBUF)
                @pl.when(j >= NBUF)
                def _():
                    k = j - NBUF
                    pltpu.make_async_copy(
                        rows.at[jbuf],
                        agg_sh.at[dstb.at[lax.rem(k // G, 2), lax.rem(k, G)]],
                        ssem.at[jbuf]).wait()
                pltpu.async_copy(
                    x_hbm.at[srcb.at[lax.rem(j // G, 2), lax.rem(j, G)]],
                    rows.at[jbuf], gsem.at[jbuf])
            return carry
        lax.fori_loop(0, n_chunk, body, 0)

        for i in range(n_chunk - NBUF, n_chunk):
            pltpu.make_async_copy(
                rows.at[i % NBUF],
                agg_sh.at[dstb.at[(i // G) % 2, i % G]],
                ssem.at[i % NBUF]).wait()
        plsc.subcore_barrier()
        pltpu.sync_copy(agg_sh.at[pl.ds(row0, nps)],
                        out_hbm.at[cid, pl.ds(row0, nps)])

    return agg_kernel(x, src, dst)


def _mlp_body(eps_ref, x_ref, agg_ref, w1_ref, b1_ref, w2_ref, b2_ref, o_ref):
    h = x_ref[...] * (1.0 + eps_ref[0]) + agg_ref[0] + agg_ref[1]
    h = jnp.dot(h, w1_ref[...], preferred_element_type=jnp.float32) + b1_ref[...]
    h = jnp.maximum(h, 0.0)
    o_ref[...] = (jnp.dot(h, w2_ref[...], preferred_element_type=jnp.float32)
                  + b2_ref[...])


def kernel(x, edge_index, W1, b1, W2, b2, eps):
    n, d = x.shape
    e = edge_index.shape[1]
    e_per_w = e // NW
    n_chunk = e_per_w // CHUNK
    npad = 10240
    nps = npad // NS

    src = edge_index[0].reshape(NW, n_chunk // G, G, CHUNK)
    dst = edge_index[1].reshape(NW, n_chunk // G, G, CHUNK)
    partial = _sc_aggregate(x, src, dst, n_chunk, npad, nps)

    br = 2000
    grid = (n // br,)
    out = pl.pallas_call(
        _mlp_body,
        grid=grid,
        in_specs=[
            pl.BlockSpec(memory_space=pltpu.SMEM),
            pl.BlockSpec((br, d), lambda i: (i, 0)),
            pl.BlockSpec((NC, br, d), lambda i: (0, i, 0)),
            pl.BlockSpec((d, d), lambda i: (0, 0)),
            pl.BlockSpec((1, d), lambda i: (0, 0)),
            pl.BlockSpec((d, d), lambda i: (0, 0)),
            pl.BlockSpec((1, d), lambda i: (0, 0)),
        ],
        out_specs=pl.BlockSpec((br, d), lambda i: (i, 0)),
        out_shape=jax.ShapeDtypeStruct((n, d), jnp.float32),
    )(eps.reshape(1), x, partial, W1, b1.reshape(1, d), W2, b2.reshape(1, d))
    return out
